# Initial kernel scaffold; baseline (speedup 1.0000x reference)
import functools

import jax
import jax.numpy as jnp
from jax import lax
from jax.experimental import pallas as pl
from jax.experimental.pallas import tpu as pltpu
from jax.experimental.pallas import tpu_sc as plsc

N_NODES = 10000
N_EDGES = 160000
D_FEAT = 256

NUM_CORES = 2
NUM_TILES = 16
HALF = D_FEAT // NUM_CORES
EDGES_PER_TILE = N_EDGES // NUM_TILES
CHUNK = 80
NUM_CHUNKS = EDGES_PER_TILE // CHUNK
ROWS_PER_TILE = N_NODES // NUM_TILES
ZROWS = 125

_mesh = plsc.VectorSubcoreMesh(core_axis_name="c", subcore_axis_name="s")


@functools.partial(
    pl.kernel,
    out_type=jax.ShapeDtypeStruct((NUM_CORES, N_NODES, HALF), jnp.float32),
    mesh=_mesh,
    scratch_types=[
        pltpu.VMEM((CHUNK,), jnp.int32),
        pltpu.VMEM((CHUNK,), jnp.int32),
        pltpu.VMEM((CHUNK, HALF), jnp.float32),
        pltpu.VMEM((ZROWS, HALF), jnp.float32),
        pltpu.VMEM_SHARED((N_NODES, HALF), jnp.float32),
        pltpu.SemaphoreType.DMA,
    ],
)
def _mp_kernel(x_lo, x_hi, edges, out, src_v, dst_v, rows_v, zbuf, acc, sem):
    cid = lax.axis_index("c")
    sid = lax.axis_index("s")

    zeros16 = jnp.zeros((16,), jnp.float32)

    @pl.loop(0, ZROWS)
    def _zero(r):
        @pl.loop(0, HALF // 16)
        def _zrow(c):
            zbuf[r, pl.ds(c * 16, 16)] = zeros16

    @pl.loop(0, ROWS_PER_TILE // ZROWS)
    def _zacc(j):
        pltpu.sync_copy(zbuf, acc.at[pl.ds(sid * ROWS_PER_TILE + j * ZROWS, ZROWS)])

    plsc.subcore_barrier()

    def edge_pass(x_half):
        base0 = sid * EDGES_PER_TILE

        @pl.loop(0, NUM_CHUNKS)
        def _step(i):
            base = base0 + i * CHUNK
            pltpu.sync_copy(edges.at[0, pl.ds(base, CHUNK)], src_v)
            pltpu.sync_copy(edges.at[1, pl.ds(base, CHUNK)], dst_v)
            pltpu.async_copy(x_half.at[src_v], rows_v, sem).wait()
            pltpu.sync_copy(rows_v, acc.at[dst_v], add=True)

    @pl.when(cid == 0)
    def _():
        edge_pass(x_lo)

    @pl.when(cid == 1)
    def _():
        edge_pass(x_hi)

    plsc.subcore_barrier()

    pltpu.sync_copy(
        acc.at[pl.ds(sid * ROWS_PER_TILE, ROWS_PER_TILE)],
        out.at[cid, pl.ds(sid * ROWS_PER_TILE, ROWS_PER_TILE)],
    )


def kernel(x, edge_index):
    x_lo = x[:, :HALF]
    x_hi = x[:, HALF:]
    out3 = _mp_kernel(x_lo, x_hi, edge_index.astype(jnp.int32))
    return jnp.transpose(out3, (1, 0, 2)).reshape(N_NODES, D_FEAT)


# speedup vs baseline: 3.6411x; 3.6411x over previous
import functools

import jax
import jax.numpy as jnp
from jax import lax
from jax.experimental import pallas as pl
from jax.experimental.pallas import tpu as pltpu
from jax.experimental.pallas import tpu_sc as plsc

N_NODES = 10000
N_EDGES = 160000
D_FEAT = 256

NUM_CORES = 2
NUM_TILES = 16
HALF = D_FEAT // NUM_CORES
EDGES_PER_TILE = N_EDGES // NUM_TILES
CHUNK = 80
NUM_CHUNKS = EDGES_PER_TILE // CHUNK
ROWS_PER_TILE = N_NODES // NUM_TILES
ZROWS = 125
WB_ROWS = 624
WB_LAST_BASE = (NUM_TILES - 1) * WB_ROWS
WB_LAST = N_NODES - WB_LAST_BASE

_mesh = plsc.VectorSubcoreMesh(core_axis_name="c", subcore_axis_name="s")


@functools.partial(
    pl.kernel,
    out_type=jax.ShapeDtypeStruct((NUM_CORES, N_NODES, HALF), jnp.float32),
    mesh=_mesh,
    scratch_types=[
        pltpu.VMEM((CHUNK,), jnp.int32),
        pltpu.VMEM((CHUNK,), jnp.int32),
        pltpu.VMEM((CHUNK, HALF), jnp.float32),
        pltpu.VMEM((ZROWS, HALF), jnp.float32),
        pltpu.VMEM_SHARED((N_NODES, HALF), jnp.float32),
        pltpu.SemaphoreType.DMA,
    ],
)
def _mp_kernel(x_lo, x_hi, src_hbm, dst_hbm, out, src_v, dst_v, rows_v, zbuf, acc, sem):
    cid = lax.axis_index("c")
    sid = lax.axis_index("s")

    zeros16 = jnp.zeros((16,), jnp.float32)

    @pl.loop(0, ZROWS)
    def _zero(r):
        @pl.loop(0, HALF // 16)
        def _zrow(c):
            zbuf[r, pl.ds(c * 16, 16)] = zeros16

    @pl.loop(0, ROWS_PER_TILE // ZROWS)
    def _zacc(j):
        pltpu.sync_copy(zbuf, acc.at[pl.ds(sid * ROWS_PER_TILE + j * ZROWS, ZROWS)])

    plsc.subcore_barrier()

    def edge_pass(x_half):
        base0 = sid * EDGES_PER_TILE

        @pl.loop(0, NUM_CHUNKS)
        def _step(i):
            base = base0 + i * CHUNK
            pltpu.sync_copy(src_hbm.at[pl.ds(base, CHUNK)], src_v)
            pltpu.sync_copy(dst_hbm.at[pl.ds(base, CHUNK)], dst_v)
            pltpu.async_copy(x_half.at[src_v], rows_v, sem).wait()
            pltpu.sync_copy(rows_v, acc.at[dst_v], add=True)

    @pl.when(cid == 0)
    def _():
        edge_pass(x_lo)

    @pl.when(cid == 1)
    def _():
        edge_pass(x_hi)

    plsc.subcore_barrier()

    @pl.when(sid < NUM_TILES - 1)
    def _wb():
        base = pl.multiple_of(sid * WB_ROWS, 8)
        pltpu.sync_copy(
            acc.at[pl.ds(base, WB_ROWS)],
            out.at[cid, pl.ds(base, WB_ROWS)],
        )

    @pl.when(sid == NUM_TILES - 1)
    def _wb_last():
        pltpu.sync_copy(
            acc.at[pl.ds(WB_LAST_BASE, WB_LAST)],
            out.at[cid, pl.ds(WB_LAST_BASE, WB_LAST)],
        )


def kernel(x, edge_index):
    x_lo = x[:, :HALF]
    x_hi = x[:, HALF:]
    ei = edge_index.astype(jnp.int32)
    out3 = _mp_kernel(x_lo, x_hi, ei[0], ei[1])
    return jnp.transpose(out3, (1, 0, 2)).reshape(N_NODES, D_FEAT)


# speedup vs baseline: 6.3693x; 1.7493x over previous
import functools

import jax
import jax.numpy as jnp
from jax import lax
from jax.experimental import pallas as pl
from jax.experimental.pallas import tpu as pltpu
from jax.experimental.pallas import tpu_sc as plsc

N_NODES = 10000
N_EDGES = 160000
D_FEAT = 256

NUM_CORES = 2
NUM_TILES = 16
HALF = D_FEAT // NUM_CORES
EDGES_PER_TILE = N_EDGES // NUM_TILES
CHUNK = 80
NUM_CHUNKS = EDGES_PER_TILE // CHUNK
ROWS_PER_TILE = N_NODES // NUM_TILES
ZROWS = 25
WB_ROWS = 624
WB_LAST_BASE = (NUM_TILES - 1) * WB_ROWS
WB_LAST = N_NODES - WB_LAST_BASE

_mesh = plsc.VectorSubcoreMesh(core_axis_name="c", subcore_axis_name="s")


@functools.partial(
    pl.kernel,
    out_type=jax.ShapeDtypeStruct((NUM_CORES, N_NODES, HALF), jnp.float32),
    mesh=_mesh,
    scratch_types=[
        pltpu.VMEM((EDGES_PER_TILE,), jnp.int32),
        pltpu.VMEM((EDGES_PER_TILE,), jnp.int32),
        pltpu.VMEM((2, CHUNK, HALF), jnp.float32),
        pltpu.VMEM((ZROWS, HALF), jnp.float32),
        pltpu.VMEM_SHARED((N_NODES, HALF), jnp.float32),
        pltpu.SemaphoreType.DMA,
        pltpu.SemaphoreType.DMA,
        pltpu.SemaphoreType.DMA,
        pltpu.SemaphoreType.DMA,
        pltpu.SemaphoreType.DMA,
    ],
)
def _mp_kernel(x_lo, x_hi, src_hbm, dst_hbm, out,
               src_all, dst_all, rows, zbuf, acc,
               sem_ld, sem_g0, sem_g1, sem_s0, sem_s1):
    cid = lax.axis_index("c")
    sid = lax.axis_index("s")
    base0 = sid * EDGES_PER_TILE

    pltpu.async_copy(src_hbm.at[pl.ds(base0, EDGES_PER_TILE)], src_all, sem_ld)
    pltpu.async_copy(dst_hbm.at[pl.ds(base0, EDGES_PER_TILE)], dst_all, sem_ld)

    zeros16 = jnp.zeros((16,), jnp.float32)

    @pl.loop(0, ZROWS)
    def _zero(r):
        @pl.loop(0, HALF // 16)
        def _zrow(c):
            zbuf[r, pl.ds(c * 16, 16)] = zeros16

    @pl.loop(0, ROWS_PER_TILE // ZROWS)
    def _zacc(j):
        pltpu.sync_copy(zbuf, acc.at[pl.ds(sid * ROWS_PER_TILE + j * ZROWS, ZROWS)])

    pltpu.make_async_copy(
        src_hbm.at[pl.ds(base0, EDGES_PER_TILE)], src_all, sem_ld).wait()
    pltpu.make_async_copy(
        dst_hbm.at[pl.ds(base0, EDGES_PER_TILE)], dst_all, sem_ld).wait()
    plsc.subcore_barrier()

    sem_g = (sem_g0, sem_g1)
    sem_s = (sem_s0, sem_s1)

    def edge_pass(xh):
        def idx(c):
            return pl.ds(pl.multiple_of(c * CHUNK, 8), CHUNK)

        def start_gather(c, b):
            pltpu.async_copy(xh.at[src_all.at[idx(c)]], rows.at[b], sem_g[b])

        def wait_gather(c, b):
            pltpu.make_async_copy(
                xh.at[src_all.at[idx(c)]], rows.at[b], sem_g[b]).wait()

        def start_scatter(c, b):
            pltpu.async_copy(
                rows.at[b], acc.at[dst_all.at[idx(c)]], sem_s[b], add=True)

        def wait_scatter(c, b):
            pltpu.make_async_copy(
                rows.at[b], acc.at[dst_all.at[idx(c)]], sem_s[b]).wait()

        start_gather(0, 0)

        @pl.loop(0, NUM_CHUNKS, step=2)
        def _step(i):
            for b in (0, 1):
                c = i + b

                @pl.when(c < NUM_CHUNKS)
                def _():
                    wait_gather(c, b)

                    @pl.when(c >= 1)
                    def _():
                        wait_scatter(c - 1, 1 - b)

                    @pl.when(c + 1 < NUM_CHUNKS)
                    def _():
                        start_gather(c + 1, 1 - b)

                    start_scatter(c, b)

        wait_scatter(NUM_CHUNKS - 1, (NUM_CHUNKS - 1) % 2)

    @pl.when(cid == 0)
    def _():
        edge_pass(x_lo)

    @pl.when(cid == 1)
    def _():
        edge_pass(x_hi)

    plsc.subcore_barrier()

    @pl.when(sid < NUM_TILES - 1)
    def _wb():
        base = pl.multiple_of(sid * WB_ROWS, 8)
        pltpu.sync_copy(
            acc.at[pl.ds(base, WB_ROWS)],
            out.at[cid, pl.ds(base, WB_ROWS)],
        )

    @pl.when(sid == NUM_TILES - 1)
    def _wb_last():
        pltpu.sync_copy(
            acc.at[pl.ds(WB_LAST_BASE, WB_LAST)],
            out.at[cid, pl.ds(WB_LAST_BASE, WB_LAST)],
        )


def kernel(x, edge_index):
    x_lo = x[:, :HALF]
    x_hi = x[:, HALF:]
    ei = edge_index.astype(jnp.int32)
    out3 = _mp_kernel(x_lo, x_hi, ei[0], ei[1])
    return jnp.transpose(out3, (1, 0, 2)).reshape(N_NODES, D_FEAT)


# speedup vs baseline: 6.9629x; 1.0932x over previous
import functools

import jax
import jax.numpy as jnp
from jax import lax
from jax.experimental import pallas as pl
from jax.experimental.pallas import tpu as pltpu
from jax.experimental.pallas import tpu_sc as plsc

N_NODES = 10000
N_EDGES = 160000
D_FEAT = 256

NUM_CORES = 2
NUM_TILES = 16
HALF = D_FEAT // NUM_CORES
EDGES_PER_TILE = N_EDGES // NUM_TILES
CHUNK = 40
NUM_CHUNKS = EDGES_PER_TILE // CHUNK
NBUF = 4
ROWS_PER_TILE = N_NODES // NUM_TILES
ZROWS = 25
WB_ROWS = 624
WB_LAST_BASE = (NUM_TILES - 1) * WB_ROWS
WB_LAST = N_NODES - WB_LAST_BASE

_mesh = plsc.VectorSubcoreMesh(core_axis_name="c", subcore_axis_name="s")


@functools.partial(
    pl.kernel,
    out_type=jax.ShapeDtypeStruct((NUM_CORES, N_NODES, HALF), jnp.float32),
    mesh=_mesh,
    scratch_types=[
        pltpu.VMEM((EDGES_PER_TILE,), jnp.int32),
        pltpu.VMEM((EDGES_PER_TILE,), jnp.int32),
        pltpu.VMEM((NBUF, CHUNK, HALF), jnp.float32),
        pltpu.VMEM((ZROWS, HALF), jnp.float32),
        pltpu.VMEM_SHARED((N_NODES, HALF), jnp.float32),
        pltpu.SemaphoreType.DMA,
        [pltpu.SemaphoreType.DMA] * NBUF,
        [pltpu.SemaphoreType.DMA] * NBUF,
    ],
)
def _mp_kernel(x_lo, x_hi, src_hbm, dst_hbm, out,
               src_all, dst_all, rows, zbuf, acc,
               sem_ld, sem_g, sem_s):
    cid = lax.axis_index("c")
    sid = lax.axis_index("s")
    base0 = sid * EDGES_PER_TILE

    pltpu.async_copy(src_hbm.at[pl.ds(base0, EDGES_PER_TILE)], src_all, sem_ld)
    pltpu.async_copy(dst_hbm.at[pl.ds(base0, EDGES_PER_TILE)], dst_all, sem_ld)

    zeros16 = jnp.zeros((16,), jnp.float32)

    @pl.loop(0, ZROWS)
    def _zero(r):
        @pl.loop(0, HALF // 16)
        def _zrow(c):
            zbuf[r, pl.ds(c * 16, 16)] = zeros16

    @pl.loop(0, ROWS_PER_TILE // ZROWS)
    def _zacc(j):
        pltpu.sync_copy(zbuf, acc.at[pl.ds(sid * ROWS_PER_TILE + j * ZROWS, ZROWS)])

    pltpu.make_async_copy(
        src_hbm.at[pl.ds(base0, EDGES_PER_TILE)], src_all, sem_ld).wait()
    pltpu.make_async_copy(
        dst_hbm.at[pl.ds(base0, EDGES_PER_TILE)], dst_all, sem_ld).wait()
    plsc.subcore_barrier()

    def edge_pass(xh):
        def idx(c):
            return pl.ds(pl.multiple_of(c * CHUNK, 8), CHUNK)

        def start_gather(c, b):
            pltpu.async_copy(xh.at[src_all.at[idx(c)]], rows.at[b], sem_g[b])

        def wait_gather(c, b):
            pltpu.make_async_copy(
                xh.at[src_all.at[idx(c)]], rows.at[b], sem_g[b]).wait()

        def start_scatter(c, b):
            pltpu.async_copy(
                rows.at[b], acc.at[dst_all.at[idx(c)]], sem_s[b], add=True)

        def wait_scatter(c, b):
            pltpu.make_async_copy(
                rows.at[b], acc.at[dst_all.at[idx(c)]], sem_s[b]).wait()

        start_gather(0, 0)
        start_gather(1, 1)

        @pl.loop(0, NUM_CHUNKS, step=NBUF)
        def _step(i):
            for b in range(NBUF):
                c = i + b
                b2 = (b + 2) % NBUF

                @pl.when(c < NUM_CHUNKS)
                def _():
                    wait_gather(c, b)

                    @pl.when(c >= 2)
                    def _():
                        wait_scatter(c - 2, b2)

                    @pl.when(c + 2 < NUM_CHUNKS)
                    def _():
                        start_gather(c + 2, b2)

                    start_scatter(c, b)

        for k in range(min(2, NUM_CHUNKS)):
            cc = NUM_CHUNKS - 1 - k
            wait_scatter(cc, cc % NBUF)

    @pl.when(cid == 0)
    def _():
        edge_pass(x_lo)

    @pl.when(cid == 1)
    def _():
        edge_pass(x_hi)

    plsc.subcore_barrier()

    @pl.when(sid < NUM_TILES - 1)
    def _wb():
        base = pl.multiple_of(sid * WB_ROWS, 8)
        pltpu.sync_copy(
            acc.at[pl.ds(base, WB_ROWS)],
            out.at[cid, pl.ds(base, WB_ROWS)],
        )

    @pl.when(sid == NUM_TILES - 1)
    def _wb_last():
        pltpu.sync_copy(
            acc.at[pl.ds(WB_LAST_BASE, WB_LAST)],
            out.at[cid, pl.ds(WB_LAST_BASE, WB_LAST)],
        )


def kernel(x, edge_index):
    x_lo = x[:, :HALF]
    x_hi = x[:, HALF:]
    ei = edge_index.astype(jnp.int32)
    out3 = _mp_kernel(x_lo, x_hi, ei[0], ei[1])
    return jnp.transpose(out3, (1, 0, 2)).reshape(N_NODES, D_FEAT)


# speedup vs baseline: 8.3692x; 1.2020x over previous
import functools

import jax
import jax.numpy as jnp
from jax import lax
from jax.experimental import pallas as pl
from jax.experimental.pallas import tpu as pltpu
from jax.experimental.pallas import tpu_sc as plsc

N_NODES = 10000
N_EDGES = 160000
D_FEAT = 256

NUM_CORES = 2
NUM_TILES = 16
HALF = D_FEAT // NUM_CORES
EDGES_PER_TILE = N_EDGES // NUM_TILES
CHUNK = 40
NUM_CHUNKS = EDGES_PER_TILE // CHUNK
NBUF = 4
ROWS_PER_TILE = N_NODES // NUM_TILES
ZROWS = 25
WB_ROWS = 624
WB_LAST_BASE = (NUM_TILES - 1) * WB_ROWS
WB_LAST = N_NODES - WB_LAST_BASE

_mesh = plsc.VectorSubcoreMesh(core_axis_name="c", subcore_axis_name="s")


@functools.partial(
    pl.kernel,
    out_type=jax.ShapeDtypeStruct((N_NODES, D_FEAT), jnp.float32),
    mesh=_mesh,
    scratch_types=[
        pltpu.VMEM((EDGES_PER_TILE,), jnp.int32),
        pltpu.VMEM((EDGES_PER_TILE,), jnp.int32),
        pltpu.VMEM((NBUF, CHUNK, HALF), jnp.float32),
        pltpu.VMEM((ZROWS, HALF), jnp.float32),
        pltpu.VMEM_SHARED((N_NODES, HALF), jnp.float32),
        pltpu.SemaphoreType.DMA,
        [pltpu.SemaphoreType.DMA] * NBUF,
        [pltpu.SemaphoreType.DMA] * NBUF,
    ],
)
def _mp_kernel(x, src_hbm, dst_hbm, out,
               src_all, dst_all, rows, zbuf, acc,
               sem_ld, sem_g, sem_s):
    cid = lax.axis_index("c")
    sid = lax.axis_index("s")
    base0 = sid * EDGES_PER_TILE
    col0 = pl.multiple_of(cid * HALF, HALF)

    pltpu.async_copy(src_hbm.at[pl.ds(base0, EDGES_PER_TILE)], src_all, sem_ld)
    pltpu.async_copy(dst_hbm.at[pl.ds(base0, EDGES_PER_TILE)], dst_all, sem_ld)

    zeros16 = jnp.zeros((16,), jnp.float32)

    @pl.loop(0, ZROWS)
    def _zero(r):
        @pl.loop(0, HALF // 16)
        def _zrow(c):
            zbuf[r, pl.ds(c * 16, 16)] = zeros16

    @pl.loop(0, ROWS_PER_TILE // ZROWS)
    def _zacc(j):
        pltpu.sync_copy(zbuf, acc.at[pl.ds(sid * ROWS_PER_TILE + j * ZROWS, ZROWS)])

    pltpu.make_async_copy(
        src_hbm.at[pl.ds(base0, EDGES_PER_TILE)], src_all, sem_ld).wait()
    pltpu.make_async_copy(
        dst_hbm.at[pl.ds(base0, EDGES_PER_TILE)], dst_all, sem_ld).wait()
    plsc.subcore_barrier()

    def edge_pass():
        def idx(c):
            return pl.ds(pl.multiple_of(c * CHUNK, 8), CHUNK)

        def start_gather(c, b):
            pltpu.async_copy(
                x.at[src_all.at[idx(c)], pl.ds(col0, HALF)], rows.at[b], sem_g[b])

        def wait_gather(c, b):
            pltpu.make_async_copy(
                x.at[src_all.at[idx(c)], pl.ds(col0, HALF)], rows.at[b],
                sem_g[b]).wait()

        def start_scatter(c, b):
            pltpu.async_copy(
                rows.at[b], acc.at[dst_all.at[idx(c)]], sem_s[b], add=True)

        def wait_scatter(c, b):
            pltpu.make_async_copy(
                rows.at[b], acc.at[dst_all.at[idx(c)]], sem_s[b]).wait()

        start_gather(0, 0)
        start_gather(1, 1)

        @pl.loop(0, NUM_CHUNKS, step=NBUF)
        def _step(i):
            for b in range(NBUF):
                c = i + b
                b2 = (b + 2) % NBUF

                @pl.when(c < NUM_CHUNKS)
                def _():
                    wait_gather(c, b)

                    @pl.when(c >= 2)
                    def _():
                        wait_scatter(c - 2, b2)

                    @pl.when(c + 2 < NUM_CHUNKS)
                    def _():
                        start_gather(c + 2, b2)

                    start_scatter(c, b)

        for k in range(min(2, NUM_CHUNKS)):
            cc = NUM_CHUNKS - 1 - k
            wait_scatter(cc, cc % NBUF)

    edge_pass()

    plsc.subcore_barrier()

    @pl.when(sid < NUM_TILES - 1)
    def _wb():
        base = pl.multiple_of(sid * WB_ROWS, 8)
        pltpu.sync_copy(
            acc.at[pl.ds(base, WB_ROWS)],
            out.at[pl.ds(base, WB_ROWS), pl.ds(col0, HALF)],
        )

    @pl.when(sid == NUM_TILES - 1)
    def _wb_last():
        pltpu.sync_copy(
            acc.at[pl.ds(WB_LAST_BASE, WB_LAST)],
            out.at[pl.ds(WB_LAST_BASE, WB_LAST), pl.ds(col0, HALF)],
        )


def kernel(x, edge_index):
    ei = edge_index.astype(jnp.int32)
    return _mp_kernel(x, ei[0], ei[1])


# speedup vs baseline: 9.8611x; 1.1783x over previous
import functools

import jax
import jax.numpy as jnp
from jax import lax
from jax.experimental import pallas as pl
from jax.experimental.pallas import tpu as pltpu
from jax.experimental.pallas import tpu_sc as plsc

N_NODES = 10000
N_EDGES = 160000
D_FEAT = 256

NUM_CORES = 2
NUM_TILES = 16
HALF = D_FEAT // NUM_CORES
EDGES_PER_TILE = N_EDGES // NUM_TILES
CHUNK = 80
NUM_CHUNKS = EDGES_PER_TILE // CHUNK
NBUF = 4
ROWS_PER_TILE = N_NODES // NUM_TILES
WB_ROWS = 624
WB_LAST_BASE = (NUM_TILES - 1) * WB_ROWS
WB_LAST = N_NODES - WB_LAST_BASE

_mesh = plsc.VectorSubcoreMesh(core_axis_name="c", subcore_axis_name="s")


@functools.partial(
    pl.kernel,
    out_type=jax.ShapeDtypeStruct((N_NODES, D_FEAT), jnp.float32),
    mesh=_mesh,
    scratch_types=[
        pltpu.VMEM((NBUF, CHUNK), jnp.int32),
        pltpu.VMEM((NBUF, CHUNK), jnp.int32),
        pltpu.VMEM((NBUF, CHUNK, HALF), jnp.float32),
        pltpu.VMEM_SHARED((N_NODES, HALF), jnp.float32),
        [pltpu.SemaphoreType.DMA] * NBUF,
        [pltpu.SemaphoreType.DMA] * NBUF,
        [pltpu.SemaphoreType.DMA] * NBUF,
        [pltpu.SemaphoreType.DMA] * NBUF,
    ],
)
def _mp_kernel(x, src_hbm, dst_hbm, out,
               src_ring, dst_ring, rows, acc,
               sem_si, sem_di, sem_g, sem_s):
    cid = lax.axis_index("c")
    sid = lax.axis_index("s")
    base0 = sid * EDGES_PER_TILE
    col0 = pl.multiple_of(cid * HALF, HALF)

    zeros16 = jnp.zeros((16,), jnp.float32)

    @pl.loop(0, CHUNK)
    def _zero(r):
        @pl.loop(0, HALF // 16)
        def _zrow(c):
            rows[0, r, pl.ds(c * 16, 16)] = zeros16

    @pl.loop(0, ROWS_PER_TILE // CHUNK)
    def _zacc(j):
        pltpu.sync_copy(
            rows.at[0], acc.at[pl.ds(sid * ROWS_PER_TILE + j * CHUNK, CHUNK)])

    _ztail = ROWS_PER_TILE - (ROWS_PER_TILE // CHUNK) * CHUNK
    pltpu.sync_copy(
        rows.at[0, pl.ds(0, _ztail)],
        acc.at[pl.ds(sid * ROWS_PER_TILE + ROWS_PER_TILE - _ztail, _ztail)])

    plsc.subcore_barrier()

    def hbm_idx(arr, c):
        return arr.at[pl.ds(pl.multiple_of(base0 + c * CHUNK, 8), CHUNK)]

    def start_src_idx(c, b):
        pltpu.async_copy(hbm_idx(src_hbm, c), src_ring.at[b], sem_si[b])

    def wait_src_idx(c, b):
        pltpu.make_async_copy(
            hbm_idx(src_hbm, c), src_ring.at[b], sem_si[b]).wait()

    def start_dst_idx(c, b):
        pltpu.async_copy(hbm_idx(dst_hbm, c), dst_ring.at[b], sem_di[b])

    def wait_dst_idx(c, b):
        pltpu.make_async_copy(
            hbm_idx(dst_hbm, c), dst_ring.at[b], sem_di[b]).wait()

    def start_gather(c, b):
        pltpu.async_copy(
            x.at[src_ring.at[b], pl.ds(col0, HALF)], rows.at[b], sem_g[b])

    def wait_gather(c, b):
        pltpu.make_async_copy(
            x.at[src_ring.at[b], pl.ds(col0, HALF)], rows.at[b],
            sem_g[b]).wait()

    def start_scatter(c, b):
        pltpu.async_copy(
            rows.at[b], acc.at[dst_ring.at[b]], sem_s[b], add=True)

    def wait_scatter(c, b):
        pltpu.make_async_copy(
            rows.at[b], acc.at[dst_ring.at[b]], sem_s[b]).wait()

    for k in range(NBUF):
        start_src_idx(k, k)
    for k in range(2):
        start_dst_idx(k, k)
    for k in range(2):
        wait_src_idx(k, k)
        start_gather(k, k)

    @pl.loop(0, NUM_CHUNKS, step=NBUF)
    def _step(i):
        for b in range(NBUF):
            c = i + b
            b2 = (b + 2) % NBUF

            @pl.when(c < NUM_CHUNKS)
            def _():
                wait_gather(c, b)

                @pl.when(c + NBUF < NUM_CHUNKS)
                def _():
                    start_src_idx(c + NBUF, b)

                @pl.when(c >= 2)
                def _():
                    wait_scatter(c - 2, b2)

                @pl.when(c + 2 < NUM_CHUNKS)
                def _():
                    start_dst_idx(c + 2, b2)
                    wait_src_idx(c + 2, b2)
                    start_gather(c + 2, b2)

                wait_dst_idx(c, b)
                start_scatter(c, b)

    for k in range(min(2, NUM_CHUNKS)):
        cc = NUM_CHUNKS - 1 - k
        wait_scatter(cc, cc % NBUF)

    plsc.subcore_barrier()

    @pl.when(sid < NUM_TILES - 1)
    def _wb():
        base = pl.multiple_of(sid * WB_ROWS, 8)
        pltpu.sync_copy(
            acc.at[pl.ds(base, WB_ROWS)],
            out.at[pl.ds(base, WB_ROWS), pl.ds(col0, HALF)],
        )

    @pl.when(sid == NUM_TILES - 1)
    def _wb_last():
        pltpu.sync_copy(
            acc.at[pl.ds(WB_LAST_BASE, WB_LAST)],
            out.at[pl.ds(WB_LAST_BASE, WB_LAST), pl.ds(col0, HALF)],
        )


def kernel(x, edge_index):
    ei = edge_index.astype(jnp.int32)
    return _mp_kernel(x, ei[0], ei[1])
